# baseline (speedup 1.0000x reference)
import functools

import jax
import jax.numpy as jnp
from jax import lax
from jax.experimental import pallas as pl
from jax.experimental.pallas import tpu as pltpu
from jax.experimental.pallas import tpu_sc as plsc

N_HEAD = 12
N_EMBD = 768
N_EXPERTS = 8
DH = N_EMBD // N_HEAD
T = 2048
TI = 256
BT = 256
NB = T // BT
FH = 4 * N_EMBD
BF = 512
NF = FH // BF
NWI = NB + N_EXPERTS - 1
_EPS = 1e-5


def _layernorm(x, g, b):
    m = jnp.mean(x, axis=-1, keepdims=True)
    v = jnp.mean((x - m) ** 2, axis=-1, keepdims=True)
    return (x - m) / jnp.sqrt(v + _EPS) * g + b


def _gelu_tanh(x):
    return 0.5 * x * (1.0 + jnp.tanh(0.7978845608028654 * (x + 0.044715 * x ** 3)))


def _mm_ln_body(x_ref, g_ref, b_ref, w_ref, bias_ref, o_ref):
    h = _layernorm(x_ref[...], g_ref[...], b_ref[...])
    o_ref[...] = (jnp.dot(h, w_ref[...], preferred_element_type=jnp.float32)
                  + bias_ref[...])


def _mm_ln(x, g, b, w, bias):
    m, k = x.shape
    n = w.shape[1]
    bm = min(BT, m)
    grid = (n // N_EMBD, m // bm)
    return pl.pallas_call(
        _mm_ln_body,
        grid=grid,
        in_specs=[
            pl.BlockSpec((bm, k), lambda ni, mi: (mi, 0)),
            pl.BlockSpec((1, k), lambda ni, mi: (0, 0)),
            pl.BlockSpec((1, k), lambda ni, mi: (0, 0)),
            pl.BlockSpec((k, N_EMBD), lambda ni, mi: (0, ni)),
            pl.BlockSpec((1, N_EMBD), lambda ni, mi: (0, ni)),
        ],
        out_specs=pl.BlockSpec((bm, N_EMBD), lambda ni, mi: (mi, ni)),
        out_shape=jax.ShapeDtypeStruct((m, n), jnp.float32),
    )(x, g.reshape(1, k), b.reshape(1, k), w, bias.reshape(1, n))


def _mm_res_body(x_ref, w_ref, bias_ref, res_ref, o_ref):
    o_ref[...] = (jnp.dot(x_ref[...], w_ref[...],
                          preferred_element_type=jnp.float32)
                  + bias_ref[...] + res_ref[...])


def _mm_res(x, w, bias, res):
    m, k = x.shape
    n = w.shape[1]
    grid = (m // BT,)
    return pl.pallas_call(
        _mm_res_body,
        grid=grid,
        in_specs=[
            pl.BlockSpec((BT, k), lambda mi: (mi, 0)),
            pl.BlockSpec((k, n), lambda mi: (0, 0)),
            pl.BlockSpec((1, n), lambda mi: (0, 0)),
            pl.BlockSpec((BT, n), lambda mi: (mi, 0)),
        ],
        out_specs=pl.BlockSpec((BT, n), lambda mi: (mi, 0)),
        out_shape=jax.ShapeDtypeStruct((m, n), jnp.float32),
    )(x, w, bias.reshape(1, n), res)


def _attn_body(q_ref, k_ref, v_ref, o_ref, *, causal, scale):
    q = q_ref[0]
    k = k_ref[0]
    v = v_ref[0]
    s = lax.dot_general(q, k, (((1,), (1,)), ((), ())),
                        preferred_element_type=jnp.float32) * scale
    if causal:
        qi = pl.program_id(1) * BT + lax.broadcasted_iota(jnp.int32, s.shape, 0)
        ki = lax.broadcasted_iota(jnp.int32, s.shape, 1)
        s = jnp.where(qi >= ki, s, -1e9)
    mx = jnp.max(s, axis=-1, keepdims=True)
    p = jnp.exp(s - mx)
    a = p / jnp.sum(p, axis=-1, keepdims=True)
    o_ref[0] = jnp.dot(a, v, preferred_element_type=jnp.float32,
                       precision=lax.Precision.HIGHEST)


def _attention(q, k, v, causal):
    h, tq, dh = q.shape
    tk = k.shape[1]
    grid = (h, tq // BT)
    body = functools.partial(_attn_body, causal=causal, scale=1.0 / (dh ** 0.5))
    return pl.pallas_call(
        body,
        grid=grid,
        in_specs=[
            pl.BlockSpec((1, BT, dh), lambda hi, qi: (hi, qi, 0)),
            pl.BlockSpec((1, tk, dh), lambda hi, qi: (hi, 0, 0)),
            pl.BlockSpec((1, tk, dh), lambda hi, qi: (hi, 0, 0)),
        ],
        out_specs=pl.BlockSpec((1, BT, dh), lambda hi, qi: (hi, qi, 0)),
        out_shape=jax.ShapeDtypeStruct((h, tq, dh), jnp.float32),
    )(q, k, v)


def _router_body(x_ref, g_ref, b_ref, wg_ref, bg_ref,
                 h3_ref, wgt_ref, eid_ref, rank_ref, cnt_ref, acc_ref):
    i = pl.program_id(0)

    @pl.when(i == 0)
    def _():
        acc_ref[...] = jnp.zeros_like(acc_ref)

    h = _layernorm(x_ref[...], g_ref[...], b_ref[...])
    h3_ref[...] = h
    logits = (jnp.dot(h, wg_ref[...], preferred_element_type=jnp.float32)
              + bg_ref[...])
    gate = jax.nn.sigmoid(logits)
    mx = jnp.max(gate, axis=-1, keepdims=True)
    cols = lax.broadcasted_iota(jnp.int32, gate.shape, 1)
    eid = jnp.min(jnp.where(gate == mx, cols, N_EXPERTS), axis=-1,
                  keepdims=True)
    onehot = (cols == eid).astype(jnp.float32)
    r = lax.broadcasted_iota(jnp.int32, (BT, BT), 0)
    c = lax.broadcasted_iota(jnp.int32, (BT, BT), 1)
    ltri = (c < r).astype(jnp.float32)
    local = jnp.dot(ltri, onehot, preferred_element_type=jnp.float32)
    acc0 = acc_ref[...]
    rank = jnp.sum(onehot * (local + acc0), axis=-1, keepdims=True)
    wgt_ref[...] = mx
    eid_ref[...] = eid.astype(jnp.float32)
    rank_ref[...] = rank
    acc_ref[...] = acc0 + jnp.sum(onehot, axis=0, keepdims=True)

    @pl.when(i == NB - 1)
    def _():
        cnt_ref[...] = acc_ref[...]


def _router(x2, g, b, wg, bg):
    grid = (NB,)
    return pl.pallas_call(
        _router_body,
        grid=grid,
        in_specs=[
            pl.BlockSpec((BT, N_EMBD), lambda i: (i, 0)),
            pl.BlockSpec((1, N_EMBD), lambda i: (0, 0)),
            pl.BlockSpec((1, N_EMBD), lambda i: (0, 0)),
            pl.BlockSpec((N_EMBD, N_EXPERTS), lambda i: (0, 0)),
            pl.BlockSpec((1, N_EXPERTS), lambda i: (0, 0)),
        ],
        out_specs=[
            pl.BlockSpec((BT, N_EMBD), lambda i: (i, 0)),
            pl.BlockSpec((BT, 1), lambda i: (i, 0)),
            pl.BlockSpec((BT, 1), lambda i: (i, 0)),
            pl.BlockSpec((BT, 1), lambda i: (i, 0)),
            pl.BlockSpec((1, N_EXPERTS), lambda i: (0, 0)),
        ],
        out_shape=[
            jax.ShapeDtypeStruct((T, N_EMBD), jnp.float32),
            jax.ShapeDtypeStruct((T, 1), jnp.float32),
            jax.ShapeDtypeStruct((T, 1), jnp.float32),
            jax.ShapeDtypeStruct((T, 1), jnp.float32),
            jax.ShapeDtypeStruct((1, N_EXPERTS), jnp.float32),
        ],
        scratch_shapes=[pltpu.VMEM((1, N_EXPERTS), jnp.float32)],
    )(x2, g.reshape(1, N_EMBD), b.reshape(1, N_EMBD), wg,
      bg.reshape(1, N_EXPERTS))


def _pos_body(eid_ref, rank_ref, starts_ref, pos_ref):
    offs = starts_ref[...]
    eid = eid_ref[...]
    cols = lax.broadcasted_iota(jnp.int32, (BT, N_EXPERTS), 1).astype(jnp.float32)
    onehot = (cols == eid).astype(jnp.float32)
    start = jnp.sum(onehot * offs, axis=1, keepdims=True)
    pos_ref[...] = (start + rank_ref[...]).astype(jnp.int32)


def _positions(eid, rank, startsf):
    return pl.pallas_call(
        _pos_body,
        grid=(NB,),
        in_specs=[
            pl.BlockSpec((BT, 1), lambda i: (i, 0)),
            pl.BlockSpec((BT, 1), lambda i: (i, 0)),
            pl.BlockSpec((1, N_EXPERTS), lambda i: (0, 0)),
        ],
        out_specs=pl.BlockSpec((BT, 1), lambda i: (i, 0)),
        out_shape=jax.ShapeDtypeStruct((T, 1), jnp.int32),
    )(eid, rank, startsf)


def _perm_body(pos_ref, perm_ref):
    i = pl.program_id(0)
    pos = pos_ref[...]
    slots = (i * BT
             + lax.broadcasted_iota(jnp.int32, (T, BT), 1))
    tok = lax.broadcasted_iota(jnp.int32, (T, BT), 0).astype(jnp.float32)
    hit = jnp.where(pos == slots, tok, 0.0)
    col = jnp.sum(hit, axis=0)
    perm_ref[...] = col.reshape(1, 1, BT).astype(jnp.int32)


def _permutation(pos):
    return pl.pallas_call(
        _perm_body,
        grid=(NB,),
        in_specs=[pl.BlockSpec((T, 1), lambda i: (0, 0))],
        out_specs=pl.BlockSpec((1, 1, BT), lambda i: (i, 0, 0)),
        out_shape=jax.ShapeDtypeStruct((NB, 1, BT), jnp.int32),
    )(pos)


def _sc_gather(table, idx):
    info = plsc.get_sparse_core_info()
    nwk = info.num_cores * info.num_subcores
    rows = T // nwk
    mesh = plsc.VectorSubcoreMesh(core_axis_name="c", subcore_axis_name="s")

    @functools.partial(
        pl.kernel,
        out_type=jax.ShapeDtypeStruct((T, N_EMBD), jnp.float32),
        mesh=mesh,
        scratch_types=[
            pltpu.VMEM((rows,), jnp.int32),
            pltpu.VMEM((rows, N_EMBD), jnp.float32),
            pltpu.SemaphoreType.DMA,
        ],
    )
    def k(table_hbm, idx_hbm, out_hbm, idx_v, rows_v, sem):
        wid = lax.axis_index("s") * info.num_cores + lax.axis_index("c")
        base = wid * rows
        pltpu.sync_copy(idx_hbm.at[pl.ds(base, rows)], idx_v)
        pltpu.async_copy(table_hbm.at[idx_v], rows_v, sem).wait()
        pltpu.sync_copy(rows_v, out_hbm.at[pl.ds(base, rows)])

    return k(table, idx)


def _moe_body(e_s, rb_s, z_s, rs_s, re_s,
              xs_ref, w1_ref, b1_ref, w2_ref, b2_ref, o_ref):
    w = pl.program_id(0)
    f = pl.program_id(1)
    x = xs_ref[...]
    h = (jnp.dot(x, w1_ref[0], preferred_element_type=jnp.float32)
         + b1_ref[0])
    h = _gelu_tanh(h)
    contrib = jnp.dot(h, w2_ref[0], preferred_element_type=jnp.float32)
    row = rb_s[w] * BT + lax.broadcasted_iota(jnp.int32, (BT, 1), 0)
    mask = jnp.logical_and(row >= rs_s[w], row < re_s[w])
    bias2 = jnp.where(f == 0, 1.0, 0.0) * b2_ref[0]
    contrib = jnp.where(mask, contrib + bias2, 0.0)
    first = jnp.logical_and(z_s[w] == 1, f == 0)

    @pl.when(first)
    def _():
        o_ref[...] = contrib

    @pl.when(jnp.logical_not(first))
    def _():
        o_ref[...] = o_ref[...] + contrib


def _moe_ffn(xs, w1, b1, w2, b2, e_arr, rb_arr, z_arr, rs_arr, re_arr):
    b1r = b1.reshape(N_EXPERTS * NF, 1, BF)
    b2r = b2.reshape(N_EXPERTS, 1, N_EMBD)
    grid_spec = pltpu.PrefetchScalarGridSpec(
        num_scalar_prefetch=5,
        grid=(NWI, NF),
        in_specs=[
            pl.BlockSpec((BT, N_EMBD),
                         lambda w, f, e_s, rb_s, z_s, rs_s, re_s: (rb_s[w], 0)),
            pl.BlockSpec((1, N_EMBD, BF),
                         lambda w, f, e_s, rb_s, z_s, rs_s, re_s: (e_s[w], 0, f)),
            pl.BlockSpec((1, 1, BF),
                         lambda w, f, e_s, rb_s, z_s, rs_s, re_s:
                         (e_s[w] * NF + f, 0, 0)),
            pl.BlockSpec((1, BF, N_EMBD),
                         lambda w, f, e_s, rb_s, z_s, rs_s, re_s: (e_s[w], f, 0)),
            pl.BlockSpec((1, 1, N_EMBD),
                         lambda w, f, e_s, rb_s, z_s, rs_s, re_s: (e_s[w], 0, 0)),
        ],
        out_specs=pl.BlockSpec(
            (BT, N_EMBD),
            lambda w, f, e_s, rb_s, z_s, rs_s, re_s: (rb_s[w], 0)),
    )
    return pl.pallas_call(
        _moe_body,
        grid_spec=grid_spec,
        out_shape=jax.ShapeDtypeStruct((T, N_EMBD), jnp.float32),
    )(e_arr, rb_arr, z_arr, rs_arr, re_arr, xs, w1, b1r, w2, b2r)


def _final_body(x_ref, m_ref, w_ref, o_ref):
    o_ref[...] = x_ref[...] + w_ref[...] * m_ref[...]


def _final_add(x2, moe, wgt):
    return pl.pallas_call(
        _final_body,
        grid=(NB,),
        in_specs=[
            pl.BlockSpec((BT, N_EMBD), lambda i: (i, 0)),
            pl.BlockSpec((BT, N_EMBD), lambda i: (i, 0)),
            pl.BlockSpec((BT, 1), lambda i: (i, 0)),
        ],
        out_specs=pl.BlockSpec((BT, N_EMBD), lambda i: (i, 0)),
        out_shape=jax.ShapeDtypeStruct((T, N_EMBD), jnp.float32),
    )(x2, moe, wgt)


def _work_items(cnt, starts):
    ends = starts + cnt
    fb = starts // BT
    lb = jnp.where(cnt > 0, (ends - 1) // BT, 0)
    nbl = jnp.where(cnt > 0, lb - fb + 1, 0)
    cum = jnp.concatenate([jnp.zeros((1,), jnp.int32), jnp.cumsum(nbl)])
    total = cum[-1]
    wi = jnp.arange(NWI, dtype=jnp.int32)
    e_arr = jnp.clip(jnp.searchsorted(cum, wi, side='right').astype(jnp.int32)
                     - 1, 0, N_EXPERTS - 1)
    valid = wi < total
    rb_arr = jnp.where(valid, fb[e_arr] + (wi - cum[e_arr]), NB - 1)
    z_arr = jnp.where(
        valid,
        jnp.concatenate([jnp.ones((1,), jnp.int32),
                         (rb_arr[1:] != rb_arr[:-1]).astype(jnp.int32)]),
        0)
    rs_arr = jnp.where(valid, starts[e_arr], 0)
    re_arr = jnp.where(valid, ends[e_arr], 0)
    return e_arr, rb_arr, z_arr, rs_arr, re_arr


def kernel(x, imgs, dis_logits, ln1_g, ln1_b, ln2_g, ln2_b, ln3_g, ln3_b,
           Wqkv, bqkv, Wproj, bproj, Wkv, bkv, Wq, bq, Wcproj, bcproj,
           Wg, bg, W1, b1, W2, b2):
    del dis_logits
    x2d = x[0]
    imgs2d = imgs[0]

    qkv = _mm_ln(x2d, ln1_g, ln1_b, Wqkv, bqkv)
    qkvh = qkv.reshape(T, 3, N_HEAD, DH).transpose(1, 2, 0, 3)
    y = _attention(qkvh[0], qkvh[1], qkvh[2], causal=True)
    y = y.transpose(1, 0, 2).reshape(T, N_EMBD)
    x1 = _mm_res(y, Wproj, bproj, x2d)

    kv = _mm_ln(imgs2d, ln2_g, ln2_b, Wkv, bkv)
    kvh = kv.reshape(TI, 2, N_HEAD, DH).transpose(1, 2, 0, 3)
    q2 = _mm_ln(x1, ln2_g, ln2_b, Wq, bq)
    q2h = q2.reshape(T, N_HEAD, DH).transpose(1, 0, 2)
    y2 = _attention(q2h, kvh[0], kvh[1], causal=False)
    y2 = y2.transpose(1, 0, 2).reshape(T, N_EMBD)
    x2v = _mm_res(y2, Wcproj, bcproj, x1)

    h3, wgt, eid, rank, counts = _router(x2v, ln3_g, ln3_b, Wg, bg)
    cnt = counts.reshape(N_EXPERTS).astype(jnp.int32)
    starts = jnp.concatenate([jnp.zeros((1,), jnp.int32),
                              jnp.cumsum(cnt)[:-1]])
    pos = _positions(eid, rank,
                     starts.astype(jnp.float32).reshape(1, N_EXPERTS))
    perm = _permutation(pos).reshape(T)
    xs = _sc_gather(h3, perm)
    wk = _work_items(cnt, starts)
    ffn_sorted = _moe_ffn(xs, W1, b1, W2, b2, *wk)
    ffn_back = _sc_gather(ffn_sorted, pos.reshape(T))
    out = _final_add(x2v, ffn_back, wgt)
    return out.reshape(1, T, N_EMBD)


# speedup vs baseline: 1.0574x; 1.0574x over previous
import functools

import jax
import jax.numpy as jnp
from jax import lax
from jax.experimental import pallas as pl
from jax.experimental.pallas import tpu as pltpu
from jax.experimental.pallas import tpu_sc as plsc

N_HEAD = 12
N_EMBD = 768
N_EXPERTS = 8
DH = N_EMBD // N_HEAD
T = 2048
TI = 256
BT = 256
NB = T // BT
FH = 4 * N_EMBD
BTM = 128
NSLOT = T + N_EXPERTS * BTM
NBM = NSLOT // BTM
NWI = T // BTM + N_EXPERTS - 1
_EPS = 1e-5


def _layernorm(x, g, b):
    m = jnp.mean(x, axis=-1, keepdims=True)
    v = jnp.mean((x - m) ** 2, axis=-1, keepdims=True)
    return (x - m) / jnp.sqrt(v + _EPS) * g + b


def _gelu_tanh(x):
    return 0.5 * x * (1.0 + jnp.tanh(0.7978845608028654 * (x + 0.044715 * x ** 3)))


def _mm_ln_body(x_ref, g_ref, b_ref, w_ref, bias_ref, o_ref):
    h = _layernorm(x_ref[...], g_ref[...], b_ref[...])
    o_ref[...] = (jnp.dot(h, w_ref[...], preferred_element_type=jnp.float32)
                  + bias_ref[...])


def _mm_ln(x, g, b, w, bias):
    m, k = x.shape
    n = w.shape[1]
    bm = min(BT, m)
    grid = (n // N_EMBD, m // bm)
    return pl.pallas_call(
        _mm_ln_body,
        grid=grid,
        in_specs=[
            pl.BlockSpec((bm, k), lambda ni, mi: (mi, 0)),
            pl.BlockSpec((1, k), lambda ni, mi: (0, 0)),
            pl.BlockSpec((1, k), lambda ni, mi: (0, 0)),
            pl.BlockSpec((k, N_EMBD), lambda ni, mi: (0, ni)),
            pl.BlockSpec((1, N_EMBD), lambda ni, mi: (0, ni)),
        ],
        out_specs=pl.BlockSpec((bm, N_EMBD), lambda ni, mi: (mi, ni)),
        out_shape=jax.ShapeDtypeStruct((m, n), jnp.float32),
    )(x, g.reshape(1, k), b.reshape(1, k), w, bias.reshape(1, n))


def _mm_res_body(x_ref, w_ref, bias_ref, res_ref, o_ref):
    o_ref[...] = (jnp.dot(x_ref[...], w_ref[...],
                          preferred_element_type=jnp.float32)
                  + bias_ref[...] + res_ref[...])


def _mm_res(x, w, bias, res):
    m, k = x.shape
    n = w.shape[1]
    grid = (m // BT,)
    return pl.pallas_call(
        _mm_res_body,
        grid=grid,
        in_specs=[
            pl.BlockSpec((BT, k), lambda mi: (mi, 0)),
            pl.BlockSpec((k, n), lambda mi: (0, 0)),
            pl.BlockSpec((1, n), lambda mi: (0, 0)),
            pl.BlockSpec((BT, n), lambda mi: (mi, 0)),
        ],
        out_specs=pl.BlockSpec((BT, n), lambda mi: (mi, 0)),
        out_shape=jax.ShapeDtypeStruct((m, n), jnp.float32),
    )(x, w, bias.reshape(1, n), res)


def _dot3x(a, b):
    bf = jnp.bfloat16
    ah = a.astype(bf)
    al = (a - ah.astype(jnp.float32)).astype(bf)
    bh = b.astype(bf)
    bl = (b - bh.astype(jnp.float32)).astype(bf)
    y = (jnp.dot(ah, bl, preferred_element_type=jnp.float32)
         + jnp.dot(al, bh, preferred_element_type=jnp.float32))
    return y + jnp.dot(ah, bh, preferred_element_type=jnp.float32)


def _causal_attn_body(q_ref, k_ref, v_ref, o_ref, *, scale):
    qi = pl.program_id(1)
    q = q_ref[0]

    def inner(j, carry):
        m_run, l_run, acc = carry
        kj = k_ref[0, pl.ds(j * BT, BT), :]
        vj = v_ref[0, pl.ds(j * BT, BT), :]
        s = lax.dot_general(q, kj, (((1,), (1,)), ((), ())),
                            preferred_element_type=jnp.float32) * scale
        r = qi * BT + lax.broadcasted_iota(jnp.int32, s.shape, 0)
        c = j * BT + lax.broadcasted_iota(jnp.int32, s.shape, 1)
        s = jnp.where(r >= c, s, -1e9)
        m_new = jnp.maximum(m_run, jnp.max(s, axis=-1, keepdims=True))
        p = jnp.exp(s - m_new)
        corr = jnp.exp(m_run - m_new)
        l_new = l_run * corr + jnp.sum(p, axis=-1, keepdims=True)
        acc_new = acc * corr + _dot3x(p, vj)
        return m_new, l_new, acc_new

    m0 = jnp.full((BT, 1), -1e30, jnp.float32)
    l0 = jnp.zeros((BT, 1), jnp.float32)
    a0 = jnp.zeros((BT, DH), jnp.float32)
    _, l, acc = lax.fori_loop(0, qi + 1, inner, (m0, l0, a0))
    o_ref[0] = acc / l


def _causal_attention(q, k, v):
    h, tq, dh = q.shape
    body = functools.partial(_causal_attn_body, scale=1.0 / (dh ** 0.5))
    return pl.pallas_call(
        body,
        grid=(h, tq // BT),
        in_specs=[
            pl.BlockSpec((1, BT, dh), lambda hi, qi: (hi, qi, 0)),
            pl.BlockSpec((1, tq, dh), lambda hi, qi: (hi, 0, 0)),
            pl.BlockSpec((1, tq, dh), lambda hi, qi: (hi, 0, 0)),
        ],
        out_specs=pl.BlockSpec((1, BT, dh), lambda hi, qi: (hi, qi, 0)),
        out_shape=jax.ShapeDtypeStruct((h, tq, dh), jnp.float32),
    )(q, k, v)


def _cross_attn_body(q_ref, k_ref, v_ref, o_ref, *, scale):
    q = q_ref[...]
    k = k_ref[...]
    v = v_ref[...]
    s = lax.dot_general(q, k, (((2,), (2,)), ((0,), (0,))),
                        preferred_element_type=jnp.float32) * scale
    mx = jnp.max(s, axis=-1, keepdims=True)
    p = jnp.exp(s - mx)
    a = p / jnp.sum(p, axis=-1, keepdims=True)
    bf = jnp.bfloat16
    ah = a.astype(bf)
    al = (a - ah.astype(jnp.float32)).astype(bf)
    vh = v.astype(bf)
    vl = (v - vh.astype(jnp.float32)).astype(bf)
    bd = (((2,), (1,)), ((0,), (0,)))
    y = (lax.dot_general(ah, vl, bd, preferred_element_type=jnp.float32)
         + lax.dot_general(al, vh, bd, preferred_element_type=jnp.float32))
    o_ref[...] = y + lax.dot_general(ah, vh, bd,
                                     preferred_element_type=jnp.float32)


def _cross_attention(q, k, v):
    h, tq, dh = q.shape
    tk = k.shape[1]
    body = functools.partial(_cross_attn_body, scale=1.0 / (dh ** 0.5))
    return pl.pallas_call(
        body,
        grid=(tq // BT,),
        in_specs=[
            pl.BlockSpec((h, BT, dh), lambda qi: (0, qi, 0)),
            pl.BlockSpec((h, tk, dh), lambda qi: (0, 0, 0)),
            pl.BlockSpec((h, tk, dh), lambda qi: (0, 0, 0)),
        ],
        out_specs=pl.BlockSpec((h, BT, dh), lambda qi: (0, qi, 0)),
        out_shape=jax.ShapeDtypeStruct((h, tq, dh), jnp.float32),
    )(q, k, v)


def _router_body(x_ref, g_ref, b_ref, wg_ref, bg_ref,
                 h3_ref, wgt_ref, eid_ref, rank_ref, cnt_ref, acc_ref):
    i = pl.program_id(0)

    @pl.when(i == 0)
    def _():
        acc_ref[...] = jnp.zeros_like(acc_ref)

    h = _layernorm(x_ref[...], g_ref[...], b_ref[...])
    h3_ref[...] = h
    logits = (jnp.dot(h, wg_ref[...], preferred_element_type=jnp.float32)
              + bg_ref[...])
    gate = jax.nn.sigmoid(logits)
    mx = jnp.max(gate, axis=-1, keepdims=True)
    cols = lax.broadcasted_iota(jnp.int32, gate.shape, 1)
    eid = jnp.min(jnp.where(gate == mx, cols, N_EXPERTS), axis=-1,
                  keepdims=True)
    onehot = (cols == eid).astype(jnp.float32)
    r = lax.broadcasted_iota(jnp.int32, (BT, BT), 0)
    c = lax.broadcasted_iota(jnp.int32, (BT, BT), 1)
    ltri = (c < r).astype(jnp.float32)
    local = jnp.dot(ltri, onehot, preferred_element_type=jnp.float32)
    acc0 = acc_ref[...]
    rank = jnp.sum(onehot * (local + acc0), axis=-1, keepdims=True)
    wgt_ref[...] = mx
    eid_ref[...] = eid.astype(jnp.float32)
    rank_ref[...] = rank
    acc_ref[...] = acc0 + jnp.sum(onehot, axis=0, keepdims=True)

    @pl.when(i == NB - 1)
    def _():
        cnt_ref[...] = acc_ref[...]


def _router(x2, g, b, wg, bg):
    grid = (NB,)
    return pl.pallas_call(
        _router_body,
        grid=grid,
        in_specs=[
            pl.BlockSpec((BT, N_EMBD), lambda i: (i, 0)),
            pl.BlockSpec((1, N_EMBD), lambda i: (0, 0)),
            pl.BlockSpec((1, N_EMBD), lambda i: (0, 0)),
            pl.BlockSpec((N_EMBD, N_EXPERTS), lambda i: (0, 0)),
            pl.BlockSpec((1, N_EXPERTS), lambda i: (0, 0)),
        ],
        out_specs=[
            pl.BlockSpec((BT, N_EMBD), lambda i: (i, 0)),
            pl.BlockSpec((BT, 1), lambda i: (i, 0)),
            pl.BlockSpec((BT, 1), lambda i: (i, 0)),
            pl.BlockSpec((BT, 1), lambda i: (i, 0)),
            pl.BlockSpec((1, N_EXPERTS), lambda i: (0, 0)),
        ],
        out_shape=[
            jax.ShapeDtypeStruct((T, N_EMBD), jnp.float32),
            jax.ShapeDtypeStruct((T, 1), jnp.float32),
            jax.ShapeDtypeStruct((T, 1), jnp.float32),
            jax.ShapeDtypeStruct((T, 1), jnp.float32),
            jax.ShapeDtypeStruct((1, N_EXPERTS), jnp.float32),
        ],
        scratch_shapes=[pltpu.VMEM((1, N_EXPERTS), jnp.float32)],
    )(x2, g.reshape(1, N_EMBD), b.reshape(1, N_EMBD), wg,
      bg.reshape(1, N_EXPERTS))


def _pos_body(eid_ref, rank_ref, starts_ref, pos_ref):
    offs = starts_ref[...]
    eid = eid_ref[...]
    cols = lax.broadcasted_iota(jnp.int32, (BT, N_EXPERTS), 1).astype(jnp.float32)
    onehot = (cols == eid).astype(jnp.float32)
    start = jnp.sum(onehot * offs, axis=1, keepdims=True)
    pos_ref[...] = (start + rank_ref[...]).astype(jnp.int32)


def _positions(eid, rank, startsf):
    return pl.pallas_call(
        _pos_body,
        grid=(NB,),
        in_specs=[
            pl.BlockSpec((BT, 1), lambda i: (i, 0)),
            pl.BlockSpec((BT, 1), lambda i: (i, 0)),
            pl.BlockSpec((1, N_EXPERTS), lambda i: (0, 0)),
        ],
        out_specs=pl.BlockSpec((BT, 1), lambda i: (i, 0)),
        out_shape=jax.ShapeDtypeStruct((T, 1), jnp.int32),
    )(eid, rank, startsf)


def _perm_body(pos_ref, perm_ref):
    i = pl.program_id(0)
    pos = pos_ref[...]
    slots = (i * BT
             + lax.broadcasted_iota(jnp.int32, (T, BT), 1))
    tok = lax.broadcasted_iota(jnp.int32, (T, BT), 0).astype(jnp.float32)
    hit = jnp.where(pos == slots, tok, 0.0)
    col = jnp.sum(hit, axis=0)
    perm_ref[...] = col.reshape(1, 1, BT).astype(jnp.int32)


def _permutation(pos):
    return pl.pallas_call(
        _perm_body,
        grid=(NSLOT // BT,),
        in_specs=[pl.BlockSpec((T, 1), lambda i: (0, 0))],
        out_specs=pl.BlockSpec((1, 1, BT), lambda i: (i, 0, 0)),
        out_shape=jax.ShapeDtypeStruct((NSLOT // BT, 1, BT), jnp.int32),
    )(pos)


def _sc_gather(table, idx):
    info = plsc.get_sparse_core_info()
    nwk = info.num_cores * info.num_subcores
    n = idx.shape[0]
    rows = n // nwk
    mesh = plsc.VectorSubcoreMesh(core_axis_name="c", subcore_axis_name="s")

    @functools.partial(
        pl.kernel,
        out_type=jax.ShapeDtypeStruct((n, N_EMBD), jnp.float32),
        mesh=mesh,
        scratch_types=[
            pltpu.VMEM((rows,), jnp.int32),
            pltpu.VMEM((rows, N_EMBD), jnp.float32),
            pltpu.SemaphoreType.DMA,
        ],
    )
    def k(table_hbm, idx_hbm, out_hbm, idx_v, rows_v, sem):
        wid = lax.axis_index("s") * info.num_cores + lax.axis_index("c")
        base = wid * rows
        pltpu.sync_copy(idx_hbm.at[pl.ds(base, rows)], idx_v)
        pltpu.async_copy(table_hbm.at[idx_v], rows_v, sem).wait()
        pltpu.sync_copy(rows_v, out_hbm.at[pl.ds(base, rows)])

    return k(table, idx)


def _moe_body(e_s, rbi_s, rbo_s, xs_ref, w1_ref, b1_ref, w2_ref, b2_ref,
              o_ref):
    x = xs_ref[...].astype(jnp.bfloat16)
    h = (jnp.dot(x, w1_ref[0], preferred_element_type=jnp.float32)
         + b1_ref[0])
    h = _gelu_tanh(h).astype(jnp.bfloat16)
    o_ref[...] = (jnp.dot(h, w2_ref[0], preferred_element_type=jnp.float32)
                  + b2_ref[0])


def _moe_ffn(xs, w1bf, b1, w2bf, b2, e_arr, rbi_arr, rbo_arr):
    b1r = b1.reshape(N_EXPERTS, 1, FH)
    b2r = b2.reshape(N_EXPERTS, 1, N_EMBD)
    grid_spec = pltpu.PrefetchScalarGridSpec(
        num_scalar_prefetch=3,
        grid=(NWI,),
        in_specs=[
            pl.BlockSpec((BTM, N_EMBD),
                         lambda w, e_s, rbi_s, rbo_s: (rbi_s[w], 0)),
            pl.BlockSpec((1, N_EMBD, FH),
                         lambda w, e_s, rbi_s, rbo_s: (e_s[w], 0, 0)),
            pl.BlockSpec((1, 1, FH),
                         lambda w, e_s, rbi_s, rbo_s: (e_s[w], 0, 0)),
            pl.BlockSpec((1, FH, N_EMBD),
                         lambda w, e_s, rbi_s, rbo_s: (e_s[w], 0, 0)),
            pl.BlockSpec((1, 1, N_EMBD),
                         lambda w, e_s, rbi_s, rbo_s: (e_s[w], 0, 0)),
        ],
        out_specs=pl.BlockSpec(
            (BTM, N_EMBD),
            lambda w, e_s, rbi_s, rbo_s: (rbo_s[w], 0)),
    )
    return pl.pallas_call(
        _moe_body,
        grid_spec=grid_spec,
        out_shape=jax.ShapeDtypeStruct(((NBM + 1) * BTM, N_EMBD), jnp.float32),
    )(e_arr, rbi_arr, rbo_arr, xs, w1bf, b1r, w2bf, b2r)


def _final_body(x_ref, m_ref, w_ref, o_ref):
    o_ref[...] = x_ref[...] + w_ref[...] * m_ref[...]


def _final_add(x2, moe, wgt):
    return pl.pallas_call(
        _final_body,
        grid=(NB,),
        in_specs=[
            pl.BlockSpec((BT, N_EMBD), lambda i: (i, 0)),
            pl.BlockSpec((BT, N_EMBD), lambda i: (i, 0)),
            pl.BlockSpec((BT, 1), lambda i: (i, 0)),
        ],
        out_specs=pl.BlockSpec((BT, N_EMBD), lambda i: (i, 0)),
        out_shape=jax.ShapeDtypeStruct((T, N_EMBD), jnp.float32),
    )(x2, moe, wgt)


def _work_items(cnt, starts_pad):
    nbl = (cnt + BTM - 1) // BTM
    cum = jnp.concatenate([jnp.zeros((1,), jnp.int32), jnp.cumsum(nbl)])
    total = cum[-1]
    wi = jnp.arange(NWI, dtype=jnp.int32)
    e_arr = jnp.clip(jnp.searchsorted(cum, wi, side='right').astype(jnp.int32)
                     - 1, 0, N_EXPERTS - 1)
    valid = wi < total
    e_last = jnp.max(jnp.where(valid, e_arr, 0))
    e_arr = jnp.where(valid, e_arr, e_last)
    rb = starts_pad[e_arr] // BTM + (wi - cum[e_arr])
    rbi_arr = jnp.where(valid, rb, 0)
    rbo_arr = jnp.where(valid, rb, NBM)
    return e_arr, rbi_arr, rbo_arr


def kernel(x, imgs, dis_logits, ln1_g, ln1_b, ln2_g, ln2_b, ln3_g, ln3_b,
           Wqkv, bqkv, Wproj, bproj, Wkv, bkv, Wq, bq, Wcproj, bcproj,
           Wg, bg, W1, b1, W2, b2):
    del dis_logits
    x2d = x[0]
    imgs2d = imgs[0]

    qkv = _mm_ln(x2d, ln1_g, ln1_b, Wqkv, bqkv)
    qkvh = qkv.reshape(T, 3, N_HEAD, DH).transpose(1, 2, 0, 3)
    y = _causal_attention(qkvh[0], qkvh[1], qkvh[2])
    y = y.transpose(1, 0, 2).reshape(T, N_EMBD)
    x1 = _mm_res(y, Wproj, bproj, x2d)

    kv = _mm_ln(imgs2d, ln2_g, ln2_b, Wkv, bkv)
    kvh = kv.reshape(TI, 2, N_HEAD, DH).transpose(1, 2, 0, 3)
    q2 = _mm_ln(x1, ln2_g, ln2_b, Wq, bq)
    q2h = q2.reshape(T, N_HEAD, DH).transpose(1, 0, 2)
    y2 = _cross_attention(q2h, kvh[0], kvh[1])
    y2 = y2.transpose(1, 0, 2).reshape(T, N_EMBD)
    x2v = _mm_res(y2, Wcproj, bcproj, x1)

    h3, wgt, eid, rank, counts = _router(x2v, ln3_g, ln3_b, Wg, bg)
    cnt = counts.reshape(N_EXPERTS).astype(jnp.int32)
    cap = ((cnt + BTM - 1) // BTM) * BTM
    starts_pad = jnp.concatenate([jnp.zeros((1,), jnp.int32),
                                  jnp.cumsum(cap)[:-1]])
    pos = _positions(eid, rank,
                     starts_pad.astype(jnp.float32).reshape(1, N_EXPERTS))
    perm = _permutation(pos).reshape(NSLOT)
    xs = _sc_gather(h3, perm)
    wk = _work_items(cnt, starts_pad)
    ffn_sorted = _moe_ffn(xs, W1.astype(jnp.bfloat16), b1,
                          W2.astype(jnp.bfloat16), b2, *wk)
    ffn_back = _sc_gather(ffn_sorted, pos.reshape(T))
    out = _final_add(x2v, ffn_back, wgt)
    return out.reshape(1, T, N_EMBD)


# speedup vs baseline: 1.3209x; 1.2492x over previous
import functools

import jax
import jax.numpy as jnp
from jax import lax
from jax.experimental import pallas as pl
from jax.experimental.pallas import tpu as pltpu
from jax.experimental.pallas import tpu_sc as plsc

N_HEAD = 12
N_EMBD = 768
N_EXPERTS = 8
DH = N_EMBD // N_HEAD
T = 2048
TI = 256
BT = 256
NB = T // BT
FH = 4 * N_EMBD
BTM = 128
NSLOT = T + N_EXPERTS * BTM
NBM = NSLOT // BTM
NWI = T // BTM + N_EXPERTS - 1
_EPS = 1e-5


def _layernorm(x, g, b):
    m = jnp.mean(x, axis=-1, keepdims=True)
    v = jnp.mean((x - m) ** 2, axis=-1, keepdims=True)
    return (x - m) / jnp.sqrt(v + _EPS) * g + b


def _gelu_tanh(x):
    return 0.5 * x * (1.0 + jnp.tanh(0.7978845608028654 * (x + 0.044715 * x ** 3)))


def _mm_ln_body(x_ref, g_ref, b_ref, w_ref, bias_ref, o_ref):
    h = _layernorm(x_ref[...], g_ref[...], b_ref[...])
    o_ref[...] = (jnp.dot(h, w_ref[...], preferred_element_type=jnp.float32)
                  + bias_ref[...])


def _mm_ln(x, g, b, w, bias):
    m, k = x.shape
    n = w.shape[1]
    bm = min(BT, m)
    return pl.pallas_call(
        _mm_ln_body,
        grid=(m // bm,),
        in_specs=[
            pl.BlockSpec((bm, k), lambda mi: (mi, 0)),
            pl.BlockSpec((1, k), lambda mi: (0, 0)),
            pl.BlockSpec((1, k), lambda mi: (0, 0)),
            pl.BlockSpec((k, n), lambda mi: (0, 0)),
            pl.BlockSpec((1, n), lambda mi: (0, 0)),
        ],
        out_specs=pl.BlockSpec((bm, n), lambda mi: (mi, 0)),
        out_shape=jax.ShapeDtypeStruct((m, n), jnp.float32),
    )(x, g.reshape(1, k), b.reshape(1, k), w, bias.reshape(1, n))


def _mm_res_body(x_ref, w_ref, bias_ref, res_ref, o_ref):
    o_ref[...] = (jnp.dot(x_ref[...], w_ref[...],
                          preferred_element_type=jnp.float32)
                  + bias_ref[...] + res_ref[...])


def _mm_res(x, w, bias, res):
    m, k = x.shape
    n = w.shape[1]
    grid = (m // BT,)
    return pl.pallas_call(
        _mm_res_body,
        grid=grid,
        in_specs=[
            pl.BlockSpec((BT, k), lambda mi: (mi, 0)),
            pl.BlockSpec((k, n), lambda mi: (0, 0)),
            pl.BlockSpec((1, n), lambda mi: (0, 0)),
            pl.BlockSpec((BT, n), lambda mi: (mi, 0)),
        ],
        out_specs=pl.BlockSpec((BT, n), lambda mi: (mi, 0)),
        out_shape=jax.ShapeDtypeStruct((m, n), jnp.float32),
    )(x, w, bias.reshape(1, n), res)


def _dot3x(a, b):
    bf = jnp.bfloat16
    ah = a.astype(bf)
    al = (a - ah.astype(jnp.float32)).astype(bf)
    bh = b.astype(bf)
    bl = (b - bh.astype(jnp.float32)).astype(bf)
    y = (jnp.dot(ah, bl, preferred_element_type=jnp.float32)
         + jnp.dot(al, bh, preferred_element_type=jnp.float32))
    return y + jnp.dot(ah, bh, preferred_element_type=jnp.float32)


BK = 512


def _causal_attn_body(q_ref, k_ref, v_ref, o_ref, *, scale):
    qi = pl.program_id(1)
    q = q_ref[0]

    def inner(j, carry):
        m_run, l_run, acc = carry
        kj = k_ref[0, pl.ds(j * BK, BK), :]
        vj = v_ref[0, pl.ds(j * BK, BK), :]
        s = lax.dot_general(q, kj, (((1,), (1,)), ((), ())),
                            preferred_element_type=jnp.float32) * scale
        r = qi * BT + lax.broadcasted_iota(jnp.int32, s.shape, 0)
        c = j * BK + lax.broadcasted_iota(jnp.int32, s.shape, 1)
        s = jnp.where(r >= c, s, -1e9)
        m_new = jnp.maximum(m_run, jnp.max(s, axis=-1, keepdims=True))
        p = jnp.exp(s - m_new)
        corr = jnp.exp(m_run - m_new)
        l_new = l_run * corr + jnp.sum(p, axis=-1, keepdims=True)
        acc_new = acc * corr + _dot3x(p, vj)
        return m_new, l_new, acc_new

    m0 = jnp.full((BT, 1), -1e30, jnp.float32)
    l0 = jnp.zeros((BT, 1), jnp.float32)
    a0 = jnp.zeros((BT, DH), jnp.float32)
    nkb = (qi * BT + BT + BK - 1) // BK
    _, l, acc = lax.fori_loop(0, nkb, inner, (m0, l0, a0))
    o_ref[0] = acc / l


def _causal_attention(q, k, v):
    h, tq, dh = q.shape
    body = functools.partial(_causal_attn_body, scale=1.0 / (dh ** 0.5))
    return pl.pallas_call(
        body,
        grid=(h, tq // BT),
        in_specs=[
            pl.BlockSpec((1, BT, dh), lambda hi, qi: (hi, qi, 0)),
            pl.BlockSpec((1, tq, dh), lambda hi, qi: (hi, 0, 0)),
            pl.BlockSpec((1, tq, dh), lambda hi, qi: (hi, 0, 0)),
        ],
        out_specs=pl.BlockSpec((1, BT, dh), lambda hi, qi: (hi, qi, 0)),
        out_shape=jax.ShapeDtypeStruct((h, tq, dh), jnp.float32),
    )(q, k, v)


def _cross_attn_body(q_ref, k_ref, v_ref, o_ref, *, scale):
    q = q_ref[...]
    k = k_ref[...]
    v = v_ref[...]
    s = lax.dot_general(q, k, (((2,), (2,)), ((0,), (0,))),
                        preferred_element_type=jnp.float32) * scale
    mx = jnp.max(s, axis=-1, keepdims=True)
    p = jnp.exp(s - mx)
    a = p / jnp.sum(p, axis=-1, keepdims=True)
    bf = jnp.bfloat16
    ah = a.astype(bf)
    al = (a - ah.astype(jnp.float32)).astype(bf)
    vh = v.astype(bf)
    vl = (v - vh.astype(jnp.float32)).astype(bf)
    bd = (((2,), (1,)), ((0,), (0,)))
    y = (lax.dot_general(ah, vl, bd, preferred_element_type=jnp.float32)
         + lax.dot_general(al, vh, bd, preferred_element_type=jnp.float32))
    o_ref[...] = y + lax.dot_general(ah, vh, bd,
                                     preferred_element_type=jnp.float32)


def _cross_attention(q, k, v):
    h, tq, dh = q.shape
    tk = k.shape[1]
    body = functools.partial(_cross_attn_body, scale=1.0 / (dh ** 0.5))
    return pl.pallas_call(
        body,
        grid=(tq // BT,),
        in_specs=[
            pl.BlockSpec((h, BT, dh), lambda qi: (0, qi, 0)),
            pl.BlockSpec((h, tk, dh), lambda qi: (0, 0, 0)),
            pl.BlockSpec((h, tk, dh), lambda qi: (0, 0, 0)),
        ],
        out_specs=pl.BlockSpec((h, BT, dh), lambda qi: (0, qi, 0)),
        out_shape=jax.ShapeDtypeStruct((h, tq, dh), jnp.float32),
    )(q, k, v)


def _router_body(x_ref, g_ref, b_ref, wg_ref, bg_ref,
                 h3_ref, wgt_ref, eid_ref, rank_ref, cnt_ref, acc_ref):
    i = pl.program_id(0)

    @pl.when(i == 0)
    def _():
        acc_ref[...] = jnp.zeros_like(acc_ref)

    h = _layernorm(x_ref[...], g_ref[...], b_ref[...])
    h3_ref[...] = h
    logits = (jnp.dot(h, wg_ref[...], preferred_element_type=jnp.float32)
              + bg_ref[...])
    gate = jax.nn.sigmoid(logits)
    mx = jnp.max(gate, axis=-1, keepdims=True)
    cols = lax.broadcasted_iota(jnp.int32, gate.shape, 1)
    eid = jnp.min(jnp.where(gate == mx, cols, N_EXPERTS), axis=-1,
                  keepdims=True)
    onehot = (cols == eid).astype(jnp.float32)
    r = lax.broadcasted_iota(jnp.int32, (BT, BT), 0)
    c = lax.broadcasted_iota(jnp.int32, (BT, BT), 1)
    ltri = (c < r).astype(jnp.float32)
    local = jnp.dot(ltri, onehot, preferred_element_type=jnp.float32)
    acc0 = acc_ref[...]
    rank = jnp.sum(onehot * (local + acc0), axis=-1, keepdims=True)
    wgt_ref[...] = mx
    eid_ref[...] = eid.astype(jnp.float32)
    rank_ref[...] = rank
    acc_ref[...] = acc0 + jnp.sum(onehot, axis=0, keepdims=True)

    @pl.when(i == NB - 1)
    def _():
        cnt_ref[...] = acc_ref[...]


def _router(x2, g, b, wg, bg):
    grid = (NB,)
    return pl.pallas_call(
        _router_body,
        grid=grid,
        in_specs=[
            pl.BlockSpec((BT, N_EMBD), lambda i: (i, 0)),
            pl.BlockSpec((1, N_EMBD), lambda i: (0, 0)),
            pl.BlockSpec((1, N_EMBD), lambda i: (0, 0)),
            pl.BlockSpec((N_EMBD, N_EXPERTS), lambda i: (0, 0)),
            pl.BlockSpec((1, N_EXPERTS), lambda i: (0, 0)),
        ],
        out_specs=[
            pl.BlockSpec((BT, N_EMBD), lambda i: (i, 0)),
            pl.BlockSpec((BT, 1), lambda i: (i, 0)),
            pl.BlockSpec((BT, 1), lambda i: (i, 0)),
            pl.BlockSpec((BT, 1), lambda i: (i, 0)),
            pl.BlockSpec((1, N_EXPERTS), lambda i: (0, 0)),
        ],
        out_shape=[
            jax.ShapeDtypeStruct((T, N_EMBD), jnp.float32),
            jax.ShapeDtypeStruct((T, 1), jnp.float32),
            jax.ShapeDtypeStruct((T, 1), jnp.float32),
            jax.ShapeDtypeStruct((T, 1), jnp.float32),
            jax.ShapeDtypeStruct((1, N_EXPERTS), jnp.float32),
        ],
        scratch_shapes=[pltpu.VMEM((1, N_EXPERTS), jnp.float32)],
    )(x2, g.reshape(1, N_EMBD), b.reshape(1, N_EMBD), wg,
      bg.reshape(1, N_EXPERTS))


def _pos_body(eid_ref, rank_ref, starts_ref, pos_ref):
    offs = starts_ref[...]
    eid = eid_ref[...]
    cols = lax.broadcasted_iota(jnp.int32, (BT, N_EXPERTS), 1).astype(jnp.float32)
    onehot = (cols == eid).astype(jnp.float32)
    start = jnp.sum(onehot * offs, axis=1, keepdims=True)
    pos_ref[...] = (start + rank_ref[...]).astype(jnp.int32)


def _positions(eid, rank, startsf):
    return pl.pallas_call(
        _pos_body,
        grid=(NB,),
        in_specs=[
            pl.BlockSpec((BT, 1), lambda i: (i, 0)),
            pl.BlockSpec((BT, 1), lambda i: (i, 0)),
            pl.BlockSpec((1, N_EXPERTS), lambda i: (0, 0)),
        ],
        out_specs=pl.BlockSpec((BT, 1), lambda i: (i, 0)),
        out_shape=jax.ShapeDtypeStruct((T, 1), jnp.int32),
    )(eid, rank, startsf)


def _perm_body(pos_ref, perm_ref):
    i = pl.program_id(0)
    pos = pos_ref[...]
    slots = (i * BT
             + lax.broadcasted_iota(jnp.int32, (T, BT), 1))
    tok = lax.broadcasted_iota(jnp.int32, (T, BT), 0).astype(jnp.float32)
    hit = jnp.where(pos == slots, tok, 0.0)
    col = jnp.sum(hit, axis=0)
    perm_ref[...] = col.reshape(1, 1, BT).astype(jnp.int32)


def _permutation(pos):
    return pl.pallas_call(
        _perm_body,
        grid=(NSLOT // BT,),
        in_specs=[pl.BlockSpec((T, 1), lambda i: (0, 0))],
        out_specs=pl.BlockSpec((1, 1, BT), lambda i: (i, 0, 0)),
        out_shape=jax.ShapeDtypeStruct((NSLOT // BT, 1, BT), jnp.int32),
    )(pos)


def _sc_gather(table, idx):
    info = plsc.get_sparse_core_info()
    nwk = info.num_cores * info.num_subcores
    n = idx.shape[0]
    rows = n // nwk
    mesh = plsc.VectorSubcoreMesh(core_axis_name="c", subcore_axis_name="s")

    @functools.partial(
        pl.kernel,
        out_type=jax.ShapeDtypeStruct((n, N_EMBD), jnp.float32),
        mesh=mesh,
        scratch_types=[
            pltpu.VMEM((rows,), jnp.int32),
            pltpu.VMEM((rows, N_EMBD), jnp.float32),
            pltpu.SemaphoreType.DMA,
        ],
    )
    def k(table_hbm, idx_hbm, out_hbm, idx_v, rows_v, sem):
        wid = lax.axis_index("s") * info.num_cores + lax.axis_index("c")
        base = wid * rows
        pltpu.sync_copy(idx_hbm.at[pl.ds(base, rows)], idx_v)
        pltpu.async_copy(table_hbm.at[idx_v], rows_v, sem).wait()
        pltpu.sync_copy(rows_v, out_hbm.at[pl.ds(base, rows)])

    return k(table, idx)


def _moe_body(e_s, rbi_s, rbo_s, xs_ref, w1_ref, b1_ref, w2_ref, b2_ref,
              o_ref):
    x = xs_ref[...]
    h = (jnp.dot(x, w1_ref[0], preferred_element_type=jnp.float32)
         + b1_ref[0])
    h = _gelu_tanh(h)
    o_ref[...] = (jnp.dot(h, w2_ref[0], preferred_element_type=jnp.float32)
                  + b2_ref[0])


def _moe_ffn(xs, w1bf, b1, w2bf, b2, e_arr, rbi_arr, rbo_arr):
    b1r = b1.reshape(N_EXPERTS, 1, FH)
    b2r = b2.reshape(N_EXPERTS, 1, N_EMBD)
    grid_spec = pltpu.PrefetchScalarGridSpec(
        num_scalar_prefetch=3,
        grid=(NWI,),
        in_specs=[
            pl.BlockSpec((BTM, N_EMBD),
                         lambda w, e_s, rbi_s, rbo_s: (rbi_s[w], 0)),
            pl.BlockSpec((1, N_EMBD, FH),
                         lambda w, e_s, rbi_s, rbo_s: (e_s[w], 0, 0)),
            pl.BlockSpec((1, 1, FH),
                         lambda w, e_s, rbi_s, rbo_s: (e_s[w], 0, 0)),
            pl.BlockSpec((1, FH, N_EMBD),
                         lambda w, e_s, rbi_s, rbo_s: (e_s[w], 0, 0)),
            pl.BlockSpec((1, 1, N_EMBD),
                         lambda w, e_s, rbi_s, rbo_s: (e_s[w], 0, 0)),
        ],
        out_specs=pl.BlockSpec(
            (BTM, N_EMBD),
            lambda w, e_s, rbi_s, rbo_s: (rbo_s[w], 0)),
    )
    return pl.pallas_call(
        _moe_body,
        grid_spec=grid_spec,
        out_shape=jax.ShapeDtypeStruct(((NBM + 1) * BTM, N_EMBD), jnp.float32),
    )(e_arr, rbi_arr, rbo_arr, xs, w1bf, b1r, w2bf, b2r)


def _final_body(x_ref, m_ref, w_ref, o_ref):
    o_ref[...] = x_ref[...] + w_ref[...] * m_ref[...]


def _final_add(x2, moe, wgt):
    return pl.pallas_call(
        _final_body,
        grid=(NB,),
        in_specs=[
            pl.BlockSpec((BT, N_EMBD), lambda i: (i, 0)),
            pl.BlockSpec((BT, N_EMBD), lambda i: (i, 0)),
            pl.BlockSpec((BT, 1), lambda i: (i, 0)),
        ],
        out_specs=pl.BlockSpec((BT, N_EMBD), lambda i: (i, 0)),
        out_shape=jax.ShapeDtypeStruct((T, N_EMBD), jnp.float32),
    )(x2, moe, wgt)


def _work_items(cnt, starts_pad):
    nbl = (cnt + BTM - 1) // BTM
    cum = jnp.concatenate([jnp.zeros((1,), jnp.int32), jnp.cumsum(nbl)])
    total = cum[-1]
    wi = jnp.arange(NWI, dtype=jnp.int32)
    e_arr = jnp.clip(jnp.searchsorted(cum, wi, side='right').astype(jnp.int32)
                     - 1, 0, N_EXPERTS - 1)
    valid = wi < total
    e_last = jnp.max(jnp.where(valid, e_arr, 0))
    e_arr = jnp.where(valid, e_arr, e_last)
    rb = starts_pad[e_arr] // BTM + (wi - cum[e_arr])
    rbi_arr = jnp.where(valid, rb, 0)
    rbo_arr = jnp.where(valid, rb, NBM)
    return e_arr, rbi_arr, rbo_arr


def kernel(x, imgs, dis_logits, ln1_g, ln1_b, ln2_g, ln2_b, ln3_g, ln3_b,
           Wqkv, bqkv, Wproj, bproj, Wkv, bkv, Wq, bq, Wcproj, bcproj,
           Wg, bg, W1, b1, W2, b2):
    del dis_logits
    x2d = x[0]
    imgs2d = imgs[0]

    qkv = _mm_ln(x2d, ln1_g, ln1_b, Wqkv, bqkv)
    qkvh = qkv.reshape(T, 3, N_HEAD, DH).transpose(1, 2, 0, 3)
    y = _causal_attention(qkvh[0], qkvh[1], qkvh[2])
    y = y.transpose(1, 0, 2).reshape(T, N_EMBD)
    x1 = _mm_res(y, Wproj, bproj, x2d)

    kv = _mm_ln(imgs2d, ln2_g, ln2_b, Wkv, bkv)
    kvh = kv.reshape(TI, 2, N_HEAD, DH).transpose(1, 2, 0, 3)
    q2 = _mm_ln(x1, ln2_g, ln2_b, Wq, bq)
    q2h = q2.reshape(T, N_HEAD, DH).transpose(1, 0, 2)
    y2 = _cross_attention(q2h, kvh[0], kvh[1])
    y2 = y2.transpose(1, 0, 2).reshape(T, N_EMBD)
    x2v = _mm_res(y2, Wcproj, bcproj, x1)

    h3, wgt, eid, rank, counts = _router(x2v, ln3_g, ln3_b, Wg, bg)
    cnt = counts.reshape(N_EXPERTS).astype(jnp.int32)
    cap = ((cnt + BTM - 1) // BTM) * BTM
    starts_pad = jnp.concatenate([jnp.zeros((1,), jnp.int32),
                                  jnp.cumsum(cap)[:-1]])
    pos = _positions(eid, rank,
                     starts_pad.astype(jnp.float32).reshape(1, N_EXPERTS))
    perm = _permutation(pos).reshape(NSLOT)
    xs = _sc_gather(h3, perm)
    wk = _work_items(cnt, starts_pad)
    ffn_sorted = _moe_ffn(xs, W1, b1, W2, b2, *wk)
    ffn_back = _sc_gather(ffn_sorted, pos.reshape(T))
    out = _final_add(x2v, ffn_back, wgt)
    return out.reshape(1, T, N_EMBD)


# speedup vs baseline: 1.6390x; 1.2408x over previous
import functools

import jax
import jax.numpy as jnp
from jax import lax
from jax.experimental import pallas as pl
from jax.experimental.pallas import tpu as pltpu
from jax.experimental.pallas import tpu_sc as plsc

N_HEAD = 12
N_EMBD = 768
N_EXPERTS = 8
DH = N_EMBD // N_HEAD
T = 2048
TI = 256
BT = 256
NB = T // BT
FH = 4 * N_EMBD
BTM = 128
NSLOT = T + N_EXPERTS * BTM
NBM = NSLOT // BTM
NWI = T // BTM + N_EXPERTS - 1
_EPS = 1e-5


def _layernorm(x, g, b):
    m = jnp.mean(x, axis=-1, keepdims=True)
    v = jnp.mean((x - m) ** 2, axis=-1, keepdims=True)
    return (x - m) / jnp.sqrt(v + _EPS) * g + b


def _gelu_tanh(x):
    return 0.5 * x * (1.0 + jnp.tanh(0.7978845608028654 * (x + 0.044715 * x ** 3)))


def _mm_ln_body(x_ref, g_ref, b_ref, w_ref, bias_ref, o_ref):
    h = _layernorm(x_ref[...], g_ref[...], b_ref[...])
    o_ref[...] = (jnp.dot(h, w_ref[...], preferred_element_type=jnp.float32)
                  + bias_ref[...])


def _mm_ln(x, g, b, w, bias):
    m, k = x.shape
    n = w.shape[1]
    bm = min(BT, m)
    return pl.pallas_call(
        _mm_ln_body,
        grid=(m // bm,),
        in_specs=[
            pl.BlockSpec((bm, k), lambda mi: (mi, 0)),
            pl.BlockSpec((1, k), lambda mi: (0, 0)),
            pl.BlockSpec((1, k), lambda mi: (0, 0)),
            pl.BlockSpec((k, n), lambda mi: (0, 0)),
            pl.BlockSpec((1, n), lambda mi: (0, 0)),
        ],
        out_specs=pl.BlockSpec((bm, n), lambda mi: (mi, 0)),
        out_shape=jax.ShapeDtypeStruct((m, n), jnp.float32),
    )(x, g.reshape(1, k), b.reshape(1, k), w, bias.reshape(1, n))


def _mm_res_body(x_ref, w_ref, bias_ref, res_ref, o_ref):
    o_ref[...] = (jnp.dot(x_ref[...], w_ref[...],
                          preferred_element_type=jnp.float32)
                  + bias_ref[...] + res_ref[...])


def _mm_res(x, w, bias, res):
    m, k = x.shape
    n = w.shape[1]
    grid = (m // BT,)
    return pl.pallas_call(
        _mm_res_body,
        grid=grid,
        in_specs=[
            pl.BlockSpec((BT, k), lambda mi: (mi, 0)),
            pl.BlockSpec((k, n), lambda mi: (0, 0)),
            pl.BlockSpec((1, n), lambda mi: (0, 0)),
            pl.BlockSpec((BT, n), lambda mi: (mi, 0)),
        ],
        out_specs=pl.BlockSpec((BT, n), lambda mi: (mi, 0)),
        out_shape=jax.ShapeDtypeStruct((m, n), jnp.float32),
    )(x, w, bias.reshape(1, n), res)


def _dot3x(a, b):
    bf = jnp.bfloat16
    ah = a.astype(bf)
    al = (a - ah.astype(jnp.float32)).astype(bf)
    bh = b.astype(bf)
    bl = (b - bh.astype(jnp.float32)).astype(bf)
    y = (jnp.dot(ah, bl, preferred_element_type=jnp.float32)
         + jnp.dot(al, bh, preferred_element_type=jnp.float32))
    return y + jnp.dot(ah, bh, preferred_element_type=jnp.float32)


BK = 512
_SCALE = 1.0 / (DH ** 0.5)


def _sm_step(q, kk, vv, off, m_run, l_run, acc, r, c):
    s = lax.dot_general(q[:, off:off + DH], kk[:, off:off + DH],
                        (((1,), (1,)), ((), ())),
                        preferred_element_type=jnp.float32) * _SCALE
    if r is not None:
        s = jnp.where(r >= c, s, -1e9)
    m_new = jnp.maximum(m_run, jnp.max(s, axis=-1, keepdims=True))
    p = jnp.exp(s - m_new)
    corr = jnp.exp(m_run - m_new)
    l_new = l_run * corr + jnp.sum(p, axis=-1, keepdims=True)
    acc_new = acc * corr + _dot3x(p, vv[:, off:off + DH])
    return m_new, l_new, acc_new


def _causal_attn_body(q_ref, k_ref, v_ref, o_ref):
    qi = pl.program_id(1)
    q = q_ref[...]

    def inner(j, carry):
        ma, la, aa, mb, lb, ab = carry
        kk = k_ref[pl.ds(j * BK, BK), :]
        vv = v_ref[pl.ds(j * BK, BK), :]
        r = qi * BT + lax.broadcasted_iota(jnp.int32, (BT, BK), 0)
        c = j * BK + lax.broadcasted_iota(jnp.int32, (BT, BK), 1)
        ma, la, aa = _sm_step(q, kk, vv, 0, ma, la, aa, r, c)
        mb, lb, ab = _sm_step(q, kk, vv, DH, mb, lb, ab, r, c)
        return ma, la, aa, mb, lb, ab

    z1 = jnp.full((BT, 1), -1e30, jnp.float32)
    z0 = jnp.zeros((BT, 1), jnp.float32)
    za = jnp.zeros((BT, DH), jnp.float32)
    nkb = (qi * BT + BT + BK - 1) // BK
    _, la, aa, _, lb, ab = lax.fori_loop(
        0, nkb, inner, (z1, z0, za, z1, z0, za))
    o_ref[...] = jnp.concatenate([aa / la, ab / lb], axis=1)


def _causal_attention(qkv):
    npair = N_HEAD // 2
    return pl.pallas_call(
        _causal_attn_body,
        grid=(npair, T // BT),
        in_specs=[
            pl.BlockSpec((BT, 2 * DH), lambda hi, qi: (qi, hi)),
            pl.BlockSpec((T, 2 * DH), lambda hi, qi: (0, npair + hi)),
            pl.BlockSpec((T, 2 * DH), lambda hi, qi: (0, 2 * npair + hi)),
        ],
        out_specs=pl.BlockSpec((BT, 2 * DH), lambda hi, qi: (qi, hi)),
        out_shape=jax.ShapeDtypeStruct((T, N_EMBD), jnp.float32),
    )(qkv, qkv, qkv)


def _cross_attn_body(q_ref, k_ref, v_ref, o_ref):
    q = q_ref[...]
    kk = k_ref[...]
    vv = v_ref[...]
    outs = []
    for off in (0, DH):
        s = lax.dot_general(q[:, off:off + DH], kk[:, off:off + DH],
                            (((1,), (1,)), ((), ())),
                            preferred_element_type=jnp.float32) * _SCALE
        mx = jnp.max(s, axis=-1, keepdims=True)
        p = jnp.exp(s - mx)
        a = p / jnp.sum(p, axis=-1, keepdims=True)
        outs.append(_dot3x(a, vv[:, off:off + DH]))
    o_ref[...] = jnp.concatenate(outs, axis=1)


def _cross_attention(q2, kv):
    npair = N_HEAD // 2
    return pl.pallas_call(
        _cross_attn_body,
        grid=(npair, T // BT),
        in_specs=[
            pl.BlockSpec((BT, 2 * DH), lambda hi, qi: (qi, hi)),
            pl.BlockSpec((TI, 2 * DH), lambda hi, qi: (0, hi)),
            pl.BlockSpec((TI, 2 * DH), lambda hi, qi: (0, npair + hi)),
        ],
        out_specs=pl.BlockSpec((BT, 2 * DH), lambda hi, qi: (qi, hi)),
        out_shape=jax.ShapeDtypeStruct((T, N_EMBD), jnp.float32),
    )(q2, kv, kv)


def _router_body(x_ref, g_ref, b_ref, wg_ref, bg_ref,
                 h3_ref, wgt_ref, eid_ref, rank_ref, cnt_ref, acc_ref):
    i = pl.program_id(0)

    @pl.when(i == 0)
    def _():
        acc_ref[...] = jnp.zeros_like(acc_ref)

    h = _layernorm(x_ref[...], g_ref[...], b_ref[...])
    h3_ref[...] = h
    logits = (jnp.dot(h, wg_ref[...], preferred_element_type=jnp.float32)
              + bg_ref[...])
    gate = jax.nn.sigmoid(logits)
    mx = jnp.max(gate, axis=-1, keepdims=True)
    cols = lax.broadcasted_iota(jnp.int32, gate.shape, 1)
    eid = jnp.min(jnp.where(gate == mx, cols, N_EXPERTS), axis=-1,
                  keepdims=True)
    onehot = (cols == eid).astype(jnp.float32)
    r = lax.broadcasted_iota(jnp.int32, (BT, BT), 0)
    c = lax.broadcasted_iota(jnp.int32, (BT, BT), 1)
    ltri = (c < r).astype(jnp.float32)
    local = jnp.dot(ltri, onehot, preferred_element_type=jnp.float32)
    acc0 = acc_ref[...]
    rank = jnp.sum(onehot * (local + acc0), axis=-1, keepdims=True)
    wgt_ref[...] = mx
    eid_ref[...] = eid.astype(jnp.float32)
    rank_ref[...] = rank
    acc_ref[...] = acc0 + jnp.sum(onehot, axis=0, keepdims=True)

    @pl.when(i == NB - 1)
    def _():
        cnt_ref[...] = acc_ref[...]


def _router(x2, g, b, wg, bg):
    grid = (NB,)
    return pl.pallas_call(
        _router_body,
        grid=grid,
        in_specs=[
            pl.BlockSpec((BT, N_EMBD), lambda i: (i, 0)),
            pl.BlockSpec((1, N_EMBD), lambda i: (0, 0)),
            pl.BlockSpec((1, N_EMBD), lambda i: (0, 0)),
            pl.BlockSpec((N_EMBD, N_EXPERTS), lambda i: (0, 0)),
            pl.BlockSpec((1, N_EXPERTS), lambda i: (0, 0)),
        ],
        out_specs=[
            pl.BlockSpec((BT, N_EMBD), lambda i: (i, 0)),
            pl.BlockSpec((BT, 1), lambda i: (i, 0)),
            pl.BlockSpec((BT, 1), lambda i: (i, 0)),
            pl.BlockSpec((BT, 1), lambda i: (i, 0)),
            pl.BlockSpec((1, N_EXPERTS), lambda i: (0, 0)),
        ],
        out_shape=[
            jax.ShapeDtypeStruct((T, N_EMBD), jnp.float32),
            jax.ShapeDtypeStruct((T, 1), jnp.float32),
            jax.ShapeDtypeStruct((T, 1), jnp.float32),
            jax.ShapeDtypeStruct((T, 1), jnp.float32),
            jax.ShapeDtypeStruct((1, N_EXPERTS), jnp.float32),
        ],
        scratch_shapes=[pltpu.VMEM((1, N_EXPERTS), jnp.float32)],
    )(x2, g.reshape(1, N_EMBD), b.reshape(1, N_EMBD), wg,
      bg.reshape(1, N_EXPERTS))


def _pos_body(eid_ref, rank_ref, starts_ref, pos_ref):
    offs = starts_ref[...]
    eid = eid_ref[...]
    cols = lax.broadcasted_iota(jnp.int32, (BT, N_EXPERTS), 1).astype(jnp.float32)
    onehot = (cols == eid).astype(jnp.float32)
    start = jnp.sum(onehot * offs, axis=1, keepdims=True)
    pos_ref[...] = (start + rank_ref[...]).astype(jnp.int32)


def _positions(eid, rank, startsf):
    return pl.pallas_call(
        _pos_body,
        grid=(NB,),
        in_specs=[
            pl.BlockSpec((BT, 1), lambda i: (i, 0)),
            pl.BlockSpec((BT, 1), lambda i: (i, 0)),
            pl.BlockSpec((1, N_EXPERTS), lambda i: (0, 0)),
        ],
        out_specs=pl.BlockSpec((BT, 1), lambda i: (i, 0)),
        out_shape=jax.ShapeDtypeStruct((T, 1), jnp.int32),
    )(eid, rank, startsf)


def _perm_body(pos_ref, perm_ref):
    i = pl.program_id(0)
    pos = pos_ref[...]
    slots = (i * BT
             + lax.broadcasted_iota(jnp.int32, (T, BT), 1))
    tok = lax.broadcasted_iota(jnp.int32, (T, BT), 0).astype(jnp.float32)
    eq = pos == slots
    hit = jnp.where(eq, tok, 0.0)
    col = jnp.sum(hit, axis=0)
    found = jnp.sum(eq.astype(jnp.float32), axis=0) > 0.0
    fallback = (slots[0] % T).astype(jnp.float32)
    col = jnp.where(found, col, fallback)
    perm_ref[...] = col.reshape(1, 1, BT).astype(jnp.int32)


def _permutation(pos):
    return pl.pallas_call(
        _perm_body,
        grid=(NSLOT // BT,),
        in_specs=[pl.BlockSpec((T, 1), lambda i: (0, 0))],
        out_specs=pl.BlockSpec((1, 1, BT), lambda i: (i, 0, 0)),
        out_shape=jax.ShapeDtypeStruct((NSLOT // BT, 1, BT), jnp.int32),
    )(pos)


def _sc_gather(table, idx):
    info = plsc.get_sparse_core_info()
    nwk = info.num_cores * info.num_subcores
    n = idx.shape[0]
    rows = n // nwk
    mesh = plsc.VectorSubcoreMesh(core_axis_name="c", subcore_axis_name="s")

    @functools.partial(
        pl.kernel,
        out_type=jax.ShapeDtypeStruct((n, N_EMBD), jnp.float32),
        mesh=mesh,
        scratch_types=[
            pltpu.VMEM((rows,), jnp.int32),
            pltpu.VMEM((rows, N_EMBD), jnp.float32),
            pltpu.SemaphoreType.DMA,
        ],
    )
    def k(table_hbm, idx_hbm, out_hbm, idx_v, rows_v, sem):
        wid = lax.axis_index("s") * info.num_cores + lax.axis_index("c")
        base = wid * rows
        pltpu.sync_copy(idx_hbm.at[pl.ds(base, rows)], idx_v)
        pltpu.async_copy(table_hbm.at[idx_v], rows_v, sem).wait()
        pltpu.sync_copy(rows_v, out_hbm.at[pl.ds(base, rows)])

    return k(table, idx)


def _moe_body(e_s, rbi_s, rbo_s, xs_ref, w1_ref, b1_ref, w2_ref, b2_ref,
              o_ref):
    x = xs_ref[...]
    h = (jnp.dot(x, w1_ref[0], preferred_element_type=jnp.float32)
         + b1_ref[0])
    h = _gelu_tanh(h)
    o_ref[...] = (jnp.dot(h, w2_ref[0], preferred_element_type=jnp.float32)
                  + b2_ref[0])


def _moe_ffn(xs, w1bf, b1, w2bf, b2, e_arr, rbi_arr, rbo_arr):
    b1r = b1.reshape(N_EXPERTS, 1, FH)
    b2r = b2.reshape(N_EXPERTS, 1, N_EMBD)
    grid_spec = pltpu.PrefetchScalarGridSpec(
        num_scalar_prefetch=3,
        grid=(NWI,),
        in_specs=[
            pl.BlockSpec((BTM, N_EMBD),
                         lambda w, e_s, rbi_s, rbo_s: (rbi_s[w], 0)),
            pl.BlockSpec((1, N_EMBD, FH),
                         lambda w, e_s, rbi_s, rbo_s: (e_s[w], 0, 0)),
            pl.BlockSpec((1, 1, FH),
                         lambda w, e_s, rbi_s, rbo_s: (e_s[w], 0, 0)),
            pl.BlockSpec((1, FH, N_EMBD),
                         lambda w, e_s, rbi_s, rbo_s: (e_s[w], 0, 0)),
            pl.BlockSpec((1, 1, N_EMBD),
                         lambda w, e_s, rbi_s, rbo_s: (e_s[w], 0, 0)),
        ],
        out_specs=pl.BlockSpec(
            (BTM, N_EMBD),
            lambda w, e_s, rbi_s, rbo_s: (rbo_s[w], 0)),
    )
    return pl.pallas_call(
        _moe_body,
        grid_spec=grid_spec,
        out_shape=jax.ShapeDtypeStruct(((NBM + 1) * BTM, N_EMBD), jnp.float32),
    )(e_arr, rbi_arr, rbo_arr, xs, w1bf, b1r, w2bf, b2r)


def _final_body(x_ref, m_ref, w_ref, o_ref):
    o_ref[...] = x_ref[...] + w_ref[...] * m_ref[...]


def _final_add(x2, moe, wgt):
    return pl.pallas_call(
        _final_body,
        grid=(NB,),
        in_specs=[
            pl.BlockSpec((BT, N_EMBD), lambda i: (i, 0)),
            pl.BlockSpec((BT, N_EMBD), lambda i: (i, 0)),
            pl.BlockSpec((BT, 1), lambda i: (i, 0)),
        ],
        out_specs=pl.BlockSpec((BT, N_EMBD), lambda i: (i, 0)),
        out_shape=jax.ShapeDtypeStruct((T, N_EMBD), jnp.float32),
    )(x2, moe, wgt)


def _work_items(cnt, starts_pad):
    nbl = (cnt + BTM - 1) // BTM
    cum = jnp.concatenate([jnp.zeros((1,), jnp.int32), jnp.cumsum(nbl)])
    total = cum[-1]
    wi = jnp.arange(NWI, dtype=jnp.int32)
    e_arr = jnp.clip(jnp.searchsorted(cum, wi, side='right').astype(jnp.int32)
                     - 1, 0, N_EXPERTS - 1)
    valid = wi < total
    e_last = jnp.max(jnp.where(valid, e_arr, 0))
    e_arr = jnp.where(valid, e_arr, e_last)
    rb = starts_pad[e_arr] // BTM + (wi - cum[e_arr])
    rbi_arr = jnp.where(valid, rb, 0)
    rbo_arr = jnp.where(valid, rb, NBM)
    return e_arr, rbi_arr, rbo_arr


def kernel(x, imgs, dis_logits, ln1_g, ln1_b, ln2_g, ln2_b, ln3_g, ln3_b,
           Wqkv, bqkv, Wproj, bproj, Wkv, bkv, Wq, bq, Wcproj, bcproj,
           Wg, bg, W1, b1, W2, b2):
    del dis_logits
    x2d = x[0]
    imgs2d = imgs[0]

    qkv = _mm_ln(x2d, ln1_g, ln1_b, Wqkv, bqkv)
    y = _causal_attention(qkv)
    x1 = _mm_res(y, Wproj, bproj, x2d)

    kv = _mm_ln(imgs2d, ln2_g, ln2_b, Wkv, bkv)
    q2 = _mm_ln(x1, ln2_g, ln2_b, Wq, bq)
    y2 = _cross_attention(q2, kv)
    x2v = _mm_res(y2, Wcproj, bcproj, x1)

    h3, wgt, eid, rank, counts = _router(x2v, ln3_g, ln3_b, Wg, bg)
    cnt = counts.reshape(N_EXPERTS).astype(jnp.int32)
    cap = ((cnt + BTM - 1) // BTM) * BTM
    starts_pad = jnp.concatenate([jnp.zeros((1,), jnp.int32),
                                  jnp.cumsum(cap)[:-1]])
    pos = _positions(eid, rank,
                     starts_pad.astype(jnp.float32).reshape(1, N_EXPERTS))
    perm = _permutation(pos).reshape(NSLOT)
    xs = _sc_gather(h3, perm)
    wk = _work_items(cnt, starts_pad)
    ffn_sorted = _moe_ffn(xs, W1, b1, W2, b2, *wk)
    ffn_back = _sc_gather(ffn_sorted, pos.reshape(T))
    out = _final_add(x2v, ffn_back, wgt)
    return out.reshape(1, T, N_EMBD)


# speedup vs baseline: 1.7655x; 1.0772x over previous
import functools

import jax
import jax.numpy as jnp
from jax import lax
from jax.experimental import pallas as pl
from jax.experimental.pallas import tpu as pltpu
from jax.experimental.pallas import tpu_sc as plsc

N_HEAD = 12
N_EMBD = 768
N_EXPERTS = 8
DH = N_EMBD // N_HEAD
T = 2048
TI = 256
BT = 256
NB = T // BT
FH = 4 * N_EMBD
BTM = 128
NSLOT = T + N_EXPERTS * BTM
NBM = NSLOT // BTM
NWI = T // BTM + N_EXPERTS - 1
_EPS = 1e-5


def _layernorm(x, g, b):
    m = jnp.mean(x, axis=-1, keepdims=True)
    v = jnp.mean((x - m) ** 2, axis=-1, keepdims=True)
    return (x - m) / jnp.sqrt(v + _EPS) * g + b


def _gelu_tanh(x):
    return 0.5 * x * (1.0 + jnp.tanh(0.7978845608028654 * (x + 0.044715 * x ** 3)))


def _mm_ln_body(x_ref, g_ref, b_ref, w_ref, bias_ref, o_ref):
    h = _layernorm(x_ref[...], g_ref[...], b_ref[...])
    o_ref[...] = (jnp.dot(h, w_ref[...], preferred_element_type=jnp.float32)
                  + bias_ref[...])


def _mm_ln(x, g, b, w, bias):
    m, k = x.shape
    n = w.shape[1]
    bm = min(BT, m)
    return pl.pallas_call(
        _mm_ln_body,
        grid=(m // bm,),
        in_specs=[
            pl.BlockSpec((bm, k), lambda mi: (mi, 0)),
            pl.BlockSpec((1, k), lambda mi: (0, 0)),
            pl.BlockSpec((1, k), lambda mi: (0, 0)),
            pl.BlockSpec((k, n), lambda mi: (0, 0)),
            pl.BlockSpec((1, n), lambda mi: (0, 0)),
        ],
        out_specs=pl.BlockSpec((bm, n), lambda mi: (mi, 0)),
        out_shape=jax.ShapeDtypeStruct((m, n), jnp.float32),
    )(x, g.reshape(1, k), b.reshape(1, k), w, bias.reshape(1, n))


def _mm_res_body(x_ref, w_ref, bias_ref, res_ref, o_ref):
    o_ref[...] = (jnp.dot(x_ref[...], w_ref[...],
                          preferred_element_type=jnp.float32)
                  + bias_ref[...] + res_ref[...])


def _mm_res(x, w, bias, res):
    m, k = x.shape
    n = w.shape[1]
    grid = (m // BT,)
    return pl.pallas_call(
        _mm_res_body,
        grid=grid,
        in_specs=[
            pl.BlockSpec((BT, k), lambda mi: (mi, 0)),
            pl.BlockSpec((k, n), lambda mi: (0, 0)),
            pl.BlockSpec((1, n), lambda mi: (0, 0)),
            pl.BlockSpec((BT, n), lambda mi: (mi, 0)),
        ],
        out_specs=pl.BlockSpec((BT, n), lambda mi: (mi, 0)),
        out_shape=jax.ShapeDtypeStruct((m, n), jnp.float32),
    )(x, w, bias.reshape(1, n), res)


def _dot3x(a, b):
    bf = jnp.bfloat16
    ah = a.astype(bf)
    al = (a - ah.astype(jnp.float32)).astype(bf)
    bh = b.astype(bf)
    bl = (b - bh.astype(jnp.float32)).astype(bf)
    y = (jnp.dot(ah, bl, preferred_element_type=jnp.float32)
         + jnp.dot(al, bh, preferred_element_type=jnp.float32))
    return y + jnp.dot(ah, bh, preferred_element_type=jnp.float32)


BK = 512
_SCALE = 1.0 / (DH ** 0.5)


def _sm_step(q, kk, vv, off, m_run, l_run, acc, r, c):
    s = lax.dot_general(q[:, off:off + DH], kk[:, off:off + DH],
                        (((1,), (1,)), ((), ())),
                        preferred_element_type=jnp.float32) * _SCALE
    if r is not None:
        s = jnp.where(r >= c, s, -1e9)
    m_new = jnp.maximum(m_run, jnp.max(s, axis=-1, keepdims=True))
    p = jnp.exp(s - m_new)
    corr = jnp.exp(m_run - m_new)
    l_new = l_run * corr + jnp.sum(p, axis=-1, keepdims=True)
    acc_new = acc * corr + _dot3x(p, vv[:, off:off + DH])
    return m_new, l_new, acc_new


def _causal_attn_body(q_ref, k_ref, v_ref, o_ref):
    qi = pl.program_id(0)
    q = q_ref[...]

    def inner(j, carry):
        kk = k_ref[pl.ds(j * BK, BK), :]
        vv = v_ref[pl.ds(j * BK, BK), :]
        r = qi * BT + lax.broadcasted_iota(jnp.int32, (BT, BK), 0)
        c = j * BK + lax.broadcasted_iota(jnp.int32, (BT, BK), 1)
        new = []
        for h in range(N_HEAD):
            m0, l0, a0 = carry[3 * h], carry[3 * h + 1], carry[3 * h + 2]
            new.extend(_sm_step(q, kk, vv, h * DH, m0, l0, a0, r, c))
        return tuple(new)

    z1 = jnp.full((BT, 1), -1e30, jnp.float32)
    z0 = jnp.zeros((BT, 1), jnp.float32)
    za = jnp.zeros((BT, DH), jnp.float32)
    init = (z1, z0, za) * N_HEAD
    nkb = (qi * BT + BT + BK - 1) // BK
    fin = lax.fori_loop(0, nkb, inner, init)
    o_ref[...] = jnp.concatenate(
        [fin[3 * h + 2] / fin[3 * h + 1] for h in range(N_HEAD)], axis=1)


def _causal_attention(qkv):
    return pl.pallas_call(
        _causal_attn_body,
        grid=(T // BT,),
        in_specs=[
            pl.BlockSpec((BT, N_EMBD), lambda qi: (qi, 0)),
            pl.BlockSpec((T, N_EMBD), lambda qi: (0, 1)),
            pl.BlockSpec((T, N_EMBD), lambda qi: (0, 2)),
        ],
        out_specs=pl.BlockSpec((BT, N_EMBD), lambda qi: (qi, 0)),
        out_shape=jax.ShapeDtypeStruct((T, N_EMBD), jnp.float32),
    )(qkv, qkv, qkv)


def _cross_attn_body(q_ref, k_ref, v_ref, o_ref):
    q = q_ref[...]
    kk = k_ref[...]
    vv = v_ref[...]
    outs = []
    for h in range(N_HEAD):
        off = h * DH
        s = lax.dot_general(q[:, off:off + DH], kk[:, off:off + DH],
                            (((1,), (1,)), ((), ())),
                            preferred_element_type=jnp.float32) * _SCALE
        mx = jnp.max(s, axis=-1, keepdims=True)
        p = jnp.exp(s - mx)
        a = p / jnp.sum(p, axis=-1, keepdims=True)
        outs.append(_dot3x(a, vv[:, off:off + DH]))
    o_ref[...] = jnp.concatenate(outs, axis=1)


def _cross_attention(q2, kv):
    return pl.pallas_call(
        _cross_attn_body,
        grid=(T // BT,),
        in_specs=[
            pl.BlockSpec((BT, N_EMBD), lambda qi: (qi, 0)),
            pl.BlockSpec((TI, N_EMBD), lambda qi: (0, 0)),
            pl.BlockSpec((TI, N_EMBD), lambda qi: (0, 1)),
        ],
        out_specs=pl.BlockSpec((BT, N_EMBD), lambda qi: (qi, 0)),
        out_shape=jax.ShapeDtypeStruct((T, N_EMBD), jnp.float32),
    )(q2, kv, kv)


def _router_body(x_ref, g_ref, b_ref, wg_ref, bg_ref,
                 h3_ref, wgt_ref, eid_ref, rank_ref, cnt_ref, acc_ref):
    i = pl.program_id(0)

    @pl.when(i == 0)
    def _():
        acc_ref[...] = jnp.zeros_like(acc_ref)

    h = _layernorm(x_ref[...], g_ref[...], b_ref[...])
    h3_ref[...] = h
    logits = (jnp.dot(h, wg_ref[...], preferred_element_type=jnp.float32)
              + bg_ref[...])
    gate = jax.nn.sigmoid(logits)
    mx = jnp.max(gate, axis=-1, keepdims=True)
    cols = lax.broadcasted_iota(jnp.int32, gate.shape, 1)
    eid = jnp.min(jnp.where(gate == mx, cols, N_EXPERTS), axis=-1,
                  keepdims=True)
    onehot = (cols == eid).astype(jnp.float32)
    r = lax.broadcasted_iota(jnp.int32, (BT, BT), 0)
    c = lax.broadcasted_iota(jnp.int32, (BT, BT), 1)
    ltri = (c < r).astype(jnp.float32)
    local = jnp.dot(ltri, onehot, preferred_element_type=jnp.float32)
    acc0 = acc_ref[...]
    rank = jnp.sum(onehot * (local + acc0), axis=-1, keepdims=True)
    wgt_ref[...] = mx
    eid_ref[...] = eid.astype(jnp.float32)
    rank_ref[...] = rank
    acc_ref[...] = acc0 + jnp.sum(onehot, axis=0, keepdims=True)

    @pl.when(i == NB - 1)
    def _():
        cnt_ref[...] = acc_ref[...]


def _router(x2, g, b, wg, bg):
    grid = (NB,)
    return pl.pallas_call(
        _router_body,
        grid=grid,
        in_specs=[
            pl.BlockSpec((BT, N_EMBD), lambda i: (i, 0)),
            pl.BlockSpec((1, N_EMBD), lambda i: (0, 0)),
            pl.BlockSpec((1, N_EMBD), lambda i: (0, 0)),
            pl.BlockSpec((N_EMBD, N_EXPERTS), lambda i: (0, 0)),
            pl.BlockSpec((1, N_EXPERTS), lambda i: (0, 0)),
        ],
        out_specs=[
            pl.BlockSpec((BT, N_EMBD), lambda i: (i, 0)),
            pl.BlockSpec((BT, 1), lambda i: (i, 0)),
            pl.BlockSpec((BT, 1), lambda i: (i, 0)),
            pl.BlockSpec((BT, 1), lambda i: (i, 0)),
            pl.BlockSpec((1, N_EXPERTS), lambda i: (0, 0)),
        ],
        out_shape=[
            jax.ShapeDtypeStruct((T, N_EMBD), jnp.float32),
            jax.ShapeDtypeStruct((T, 1), jnp.float32),
            jax.ShapeDtypeStruct((T, 1), jnp.float32),
            jax.ShapeDtypeStruct((T, 1), jnp.float32),
            jax.ShapeDtypeStruct((1, N_EXPERTS), jnp.float32),
        ],
        scratch_shapes=[pltpu.VMEM((1, N_EXPERTS), jnp.float32)],
    )(x2, g.reshape(1, N_EMBD), b.reshape(1, N_EMBD), wg,
      bg.reshape(1, N_EXPERTS))


def _pos_body(eid_ref, rank_ref, starts_ref, pos_ref):
    offs = starts_ref[...]
    eid = eid_ref[...]
    cols = lax.broadcasted_iota(jnp.int32, (BT, N_EXPERTS), 1).astype(jnp.float32)
    onehot = (cols == eid).astype(jnp.float32)
    start = jnp.sum(onehot * offs, axis=1, keepdims=True)
    pos_ref[...] = (start + rank_ref[...]).astype(jnp.int32)


def _positions(eid, rank, startsf):
    return pl.pallas_call(
        _pos_body,
        grid=(NB,),
        in_specs=[
            pl.BlockSpec((BT, 1), lambda i: (i, 0)),
            pl.BlockSpec((BT, 1), lambda i: (i, 0)),
            pl.BlockSpec((1, N_EXPERTS), lambda i: (0, 0)),
        ],
        out_specs=pl.BlockSpec((BT, 1), lambda i: (i, 0)),
        out_shape=jax.ShapeDtypeStruct((T, 1), jnp.int32),
    )(eid, rank, startsf)


def _perm_body(pos_ref, perm_ref):
    i = pl.program_id(0)
    pos = pos_ref[...]
    slots = (i * BT
             + lax.broadcasted_iota(jnp.int32, (T, BT), 1))
    tok = lax.broadcasted_iota(jnp.int32, (T, BT), 0).astype(jnp.float32)
    eq = pos == slots
    hit = jnp.where(eq, tok, 0.0)
    col = jnp.sum(hit, axis=0)
    found = jnp.sum(eq.astype(jnp.float32), axis=0) > 0.0
    fallback = (slots[0] % T).astype(jnp.float32)
    col = jnp.where(found, col, fallback)
    perm_ref[...] = col.reshape(1, 1, BT).astype(jnp.int32)


def _permutation(pos):
    return pl.pallas_call(
        _perm_body,
        grid=(NSLOT // BT,),
        in_specs=[pl.BlockSpec((T, 1), lambda i: (0, 0))],
        out_specs=pl.BlockSpec((1, 1, BT), lambda i: (i, 0, 0)),
        out_shape=jax.ShapeDtypeStruct((NSLOT // BT, 1, BT), jnp.int32),
    )(pos)


def _sc_gather(table, idx):
    info = plsc.get_sparse_core_info()
    nwk = info.num_cores * info.num_subcores
    n = idx.shape[0]
    rows = n // nwk
    mesh = plsc.VectorSubcoreMesh(core_axis_name="c", subcore_axis_name="s")

    @functools.partial(
        pl.kernel,
        out_type=jax.ShapeDtypeStruct((n, N_EMBD), jnp.float32),
        mesh=mesh,
        scratch_types=[
            pltpu.VMEM((rows,), jnp.int32),
            pltpu.VMEM((rows, N_EMBD), jnp.float32),
            pltpu.SemaphoreType.DMA,
        ],
    )
    def k(table_hbm, idx_hbm, out_hbm, idx_v, rows_v, sem):
        wid = lax.axis_index("s") * info.num_cores + lax.axis_index("c")
        base = wid * rows
        pltpu.sync_copy(idx_hbm.at[pl.ds(base, rows)], idx_v)
        pltpu.async_copy(table_hbm.at[idx_v], rows_v, sem).wait()
        pltpu.sync_copy(rows_v, out_hbm.at[pl.ds(base, rows)])

    return k(table, idx)


def _moe_body(e_s, rbi_s, rbo_s, xs_ref, w1_ref, b1_ref, w2_ref, b2_ref,
              o_ref):
    x = xs_ref[...]
    h = (jnp.dot(x, w1_ref[0], preferred_element_type=jnp.float32)
         + b1_ref[0])
    h = _gelu_tanh(h)
    o_ref[...] = (jnp.dot(h, w2_ref[0], preferred_element_type=jnp.float32)
                  + b2_ref[0])


def _moe_ffn(xs, w1bf, b1, w2bf, b2, e_arr, rbi_arr, rbo_arr):
    b1r = b1.reshape(N_EXPERTS, 1, FH)
    b2r = b2.reshape(N_EXPERTS, 1, N_EMBD)
    grid_spec = pltpu.PrefetchScalarGridSpec(
        num_scalar_prefetch=3,
        grid=(NWI,),
        in_specs=[
            pl.BlockSpec((BTM, N_EMBD),
                         lambda w, e_s, rbi_s, rbo_s: (rbi_s[w], 0)),
            pl.BlockSpec((1, N_EMBD, FH),
                         lambda w, e_s, rbi_s, rbo_s: (e_s[w], 0, 0)),
            pl.BlockSpec((1, 1, FH),
                         lambda w, e_s, rbi_s, rbo_s: (e_s[w], 0, 0)),
            pl.BlockSpec((1, FH, N_EMBD),
                         lambda w, e_s, rbi_s, rbo_s: (e_s[w], 0, 0)),
            pl.BlockSpec((1, 1, N_EMBD),
                         lambda w, e_s, rbi_s, rbo_s: (e_s[w], 0, 0)),
        ],
        out_specs=pl.BlockSpec(
            (BTM, N_EMBD),
            lambda w, e_s, rbi_s, rbo_s: (rbo_s[w], 0)),
    )
    return pl.pallas_call(
        _moe_body,
        grid_spec=grid_spec,
        out_shape=jax.ShapeDtypeStruct(((NBM + 1) * BTM, N_EMBD), jnp.float32),
    )(e_arr, rbi_arr, rbo_arr, xs, w1bf, b1r, w2bf, b2r)


def _final_body(x_ref, m_ref, w_ref, o_ref):
    o_ref[...] = x_ref[...] + w_ref[...] * m_ref[...]


def _final_add(x2, moe, wgt):
    return pl.pallas_call(
        _final_body,
        grid=(NB,),
        in_specs=[
            pl.BlockSpec((BT, N_EMBD), lambda i: (i, 0)),
            pl.BlockSpec((BT, N_EMBD), lambda i: (i, 0)),
            pl.BlockSpec((BT, 1), lambda i: (i, 0)),
        ],
        out_specs=pl.BlockSpec((BT, N_EMBD), lambda i: (i, 0)),
        out_shape=jax.ShapeDtypeStruct((T, N_EMBD), jnp.float32),
    )(x2, moe, wgt)


def _work_items(cnt, starts_pad):
    nbl = (cnt + BTM - 1) // BTM
    cum = jnp.concatenate([jnp.zeros((1,), jnp.int32), jnp.cumsum(nbl)])
    total = cum[-1]
    wi = jnp.arange(NWI, dtype=jnp.int32)
    e_arr = jnp.clip(jnp.searchsorted(cum, wi, side='right').astype(jnp.int32)
                     - 1, 0, N_EXPERTS - 1)
    valid = wi < total
    e_last = jnp.max(jnp.where(valid, e_arr, 0))
    e_arr = jnp.where(valid, e_arr, e_last)
    rb = starts_pad[e_arr] // BTM + (wi - cum[e_arr])
    rbi_arr = jnp.where(valid, rb, 0)
    rbo_arr = jnp.where(valid, rb, NBM)
    return e_arr, rbi_arr, rbo_arr


def kernel(x, imgs, dis_logits, ln1_g, ln1_b, ln2_g, ln2_b, ln3_g, ln3_b,
           Wqkv, bqkv, Wproj, bproj, Wkv, bkv, Wq, bq, Wcproj, bcproj,
           Wg, bg, W1, b1, W2, b2):
    del dis_logits
    x2d = x[0]
    imgs2d = imgs[0]

    qkv = _mm_ln(x2d, ln1_g, ln1_b, Wqkv, bqkv)
    y = _causal_attention(qkv)
    x1 = _mm_res(y, Wproj, bproj, x2d)

    kv = _mm_ln(imgs2d, ln2_g, ln2_b, Wkv, bkv)
    q2 = _mm_ln(x1, ln2_g, ln2_b, Wq, bq)
    y2 = _cross_attention(q2, kv)
    x2v = _mm_res(y2, Wcproj, bcproj, x1)

    h3, wgt, eid, rank, counts = _router(x2v, ln3_g, ln3_b, Wg, bg)
    cnt = counts.reshape(N_EXPERTS).astype(jnp.int32)
    cap = ((cnt + BTM - 1) // BTM) * BTM
    starts_pad = jnp.concatenate([jnp.zeros((1,), jnp.int32),
                                  jnp.cumsum(cap)[:-1]])
    pos = _positions(eid, rank,
                     starts_pad.astype(jnp.float32).reshape(1, N_EXPERTS))
    perm = _permutation(pos).reshape(NSLOT)
    xs = _sc_gather(h3, perm)
    wk = _work_items(cnt, starts_pad)
    ffn_sorted = _moe_ffn(xs, W1, b1, W2, b2, *wk)
    ffn_back = _sc_gather(ffn_sorted, pos.reshape(T))
    out = _final_add(x2v, ffn_back, wgt)
    return out.reshape(1, T, N_EMBD)
